# baseline (speedup 1.0000x reference)
import functools

import jax
import jax.numpy as jnp
from jax import lax
from jax.experimental import pallas as pl
from jax.experimental.pallas import tpu as pltpu
from jax.experimental.pallas import tpu_sc as plsc

_BATCH = 4096
_HIST = 50
_BT = _BATCH * _HIST
_D = 32
_DOUT = 128

_NC = 2
_NS = 16
_NW = _NC * _NS
_BPW = _BT // _NW
_IROWS = _BPW // 128
_CH = 1600
_NCH = _BPW // _CH


def _sc_gather(idx128, table):
    mesh = plsc.VectorSubcoreMesh(core_axis_name="c", subcore_axis_name="s")

    @functools.partial(
        pl.kernel,
        out_type=jax.ShapeDtypeStruct((_BT, _D), jnp.float32),
        mesh=mesh,
        scratch_types=[
            pltpu.VMEM((_BPW,), jnp.int32),
            pltpu.VMEM((_CH, _D), jnp.float32),
            pltpu.SemaphoreType.DMA,
        ],
        compiler_params=pltpu.CompilerParams(
            use_tc_tiling_on_sc=False, needs_layout_passes=False
        ),
    )
    def gather_kernel(idx_hbm, table_hbm, out_hbm, idxf_v, rows_v, sem):
        wid = lax.axis_index("s") * _NC + lax.axis_index("c")
        base = wid * _BPW
        for r in range(_IROWS):
            pltpu.sync_copy(
                idx_hbm.at[wid * _IROWS + r], idxf_v.at[pl.ds(r * 128, 128)]
            )
        for ch in range(_NCH):
            off = base + ch * _CH
            pltpu.async_copy(
                table_hbm.at[idxf_v.at[pl.ds(ch * _CH, _CH)]], rows_v, sem
            ).wait()
            pltpu.sync_copy(rows_v, out_hbm.at[pl.ds(off, _CH)])

    return gather_kernel(idx128, table)


def _tc_project(gathered, project_kernel):
    blk_b = 64

    def mm_body(g_ref, p_ref, o_ref):
        res = jnp.dot(g_ref[...], p_ref[...], preferred_element_type=jnp.float32)
        o_ref[...] = res.reshape(blk_b, _HIST, _DOUT)

    return pl.pallas_call(
        mm_body,
        grid=(_BATCH // blk_b,),
        in_specs=[
            pl.BlockSpec((blk_b * _HIST, _D), lambda i: (i, 0)),
            pl.BlockSpec((_D, _DOUT), lambda i: (0, 0)),
        ],
        out_specs=pl.BlockSpec((blk_b, _HIST, _DOUT), lambda i: (i, 0, 0)),
        out_shape=jax.ShapeDtypeStruct((_BATCH, _HIST, _DOUT), jnp.float32),
    )(gathered, project_kernel)


def kernel(inputs, embeddings, project_kernel):
    idx128 = jnp.reshape(inputs.astype(jnp.int32), (_BT // 128, 128))
    gathered = _sc_gather(idx128, embeddings)
    return _tc_project(gathered, project_kernel)


# speedup vs baseline: 1.1185x; 1.1185x over previous
import functools

import jax
import jax.numpy as jnp
from jax import lax
from jax.experimental import pallas as pl
from jax.experimental.pallas import tpu as pltpu
from jax.experimental.pallas import tpu_sc as plsc

_BATCH = 4096
_HIST = 50
_BT = _BATCH * _HIST
_D = 32
_DOUT = 128

_NC = 2
_NS = 16
_NW = _NC * _NS
_BPW = _BT // _NW
_BROWS = _BATCH // _NW
_CH = 1600
_NCH = _BPW // _CH
_L = 16
_MAGIC, _SHIFT = 5243, 18


def _sc_gather(idx_pad, table):
    mesh = plsc.VectorSubcoreMesh(core_axis_name="c", subcore_axis_name="s")

    @functools.partial(
        pl.kernel,
        out_type=jax.ShapeDtypeStruct((_BT, _DOUT), jnp.float32),
        mesh=mesh,
        scratch_types=[
            pltpu.VMEM((_BROWS, _DOUT), jnp.int32),
            pltpu.VMEM((_BPW,), jnp.int32),
            pltpu.VMEM((_CH, _D), jnp.float32),
            pltpu.SemaphoreType.DMA,
        ],
        compiler_params=pltpu.CompilerParams(
            use_tc_tiling_on_sc=False, needs_layout_passes=False
        ),
    )
    def gather_kernel(idx_hbm, table_hbm, out_hbm, idx2_v, idxf_v, rows_v, sem):
        wid = lax.axis_index("s") * _NC + lax.axis_index("c")
        base = wid * _BPW
        pltpu.sync_copy(idx_hbm.at[pl.ds(wid * _BROWS, _BROWS)], idx2_v)

        lanes = lax.iota(jnp.int32, _L)
        for g in range(_BPW // _L):
            jv = g * _L + lanes
            r = lax.shift_right_logical(jv * _MAGIC, _SHIFT)
            c = jv - r * _HIST
            idxf_v[pl.ds(g * _L, _L)] = plsc.load_gather(idx2_v, [r, c])

        for ch in range(_NCH):
            off = base + ch * _CH
            pltpu.async_copy(
                table_hbm.at[idxf_v.at[pl.ds(ch * _CH, _CH)]], rows_v, sem
            ).wait()
            pltpu.sync_copy(rows_v, out_hbm.at[pl.ds(off, _CH), pl.ds(0, _D)])

    return gather_kernel(idx_pad, table)


def _tc_project(gathered, project_kernel):
    blk_b = 64

    def mm_body(g_ref, p_ref, o_ref):
        g = g_ref[...][:, : _D]
        res = jnp.dot(g, p_ref[...], preferred_element_type=jnp.float32)
        o_ref[...] = res.reshape(blk_b, _HIST, _DOUT)

    return pl.pallas_call(
        mm_body,
        grid=(_BATCH // blk_b,),
        in_specs=[
            pl.BlockSpec((blk_b * _HIST, _DOUT), lambda i: (i, 0)),
            pl.BlockSpec((_D, _DOUT), lambda i: (0, 0)),
        ],
        out_specs=pl.BlockSpec((blk_b, _HIST, _DOUT), lambda i: (i, 0, 0)),
        out_shape=jax.ShapeDtypeStruct((_BATCH, _HIST, _DOUT), jnp.float32),
    )(gathered, project_kernel)


def kernel(inputs, embeddings, project_kernel):
    idx_pad = jnp.pad(inputs.astype(jnp.int32), ((0, 0), (0, _DOUT - _HIST)))
    gathered = _sc_gather(idx_pad, embeddings)
    return _tc_project(gathered, project_kernel)


# speedup vs baseline: 1.4389x; 1.2865x over previous
import functools

import jax
import jax.numpy as jnp
from jax import lax
from jax.experimental import pallas as pl
from jax.experimental.pallas import tpu as pltpu
from jax.experimental.pallas import tpu_sc as plsc

_BATCH = 4096
_HIST = 50
_BT = _BATCH * _HIST
_V = 1000000
_D = 32
_DOUT = 128

_NC = 2
_NS = 16
_NW = _NC * _NS
_BPW = _BT // _NW
_BROWS = _BATCH // _NW
_CH = 640
_NCH = _BPW // _CH
_L = 16
_MAGIC, _SHIFT = 5243, 18


def _tc_project_table(embeddings, project_kernel):
    tableT = jnp.transpose(embeddings)
    blk = 8192

    def body(tT_ref, p_ref, o_ref):
        o_ref[...] = jax.lax.dot_general(
            tT_ref[...], p_ref[...],
            dimension_numbers=(((0,), (0,)), ((), ())),
            preferred_element_type=jnp.float32,
        )

    return pl.pallas_call(
        body,
        grid=(pl.cdiv(_V, blk),),
        in_specs=[
            pl.BlockSpec((_D, blk), lambda i: (0, i)),
            pl.BlockSpec((_D, _DOUT), lambda i: (0, 0)),
        ],
        out_specs=pl.BlockSpec((blk, _DOUT), lambda i: (i, 0)),
        out_shape=jax.ShapeDtypeStruct((_V, _DOUT), jnp.float32),
    )(tableT, project_kernel)


def _sc_gather(idx_pad, proj):
    mesh = plsc.VectorSubcoreMesh(core_axis_name="c", subcore_axis_name="s")

    @functools.partial(
        pl.kernel,
        out_type=jax.ShapeDtypeStruct((_BT, _DOUT), jnp.float32),
        mesh=mesh,
        scratch_types=[
            pltpu.VMEM((_BROWS, _DOUT), jnp.int32),
            pltpu.VMEM((_BPW,), jnp.int32),
            pltpu.VMEM((_CH, _DOUT), jnp.float32),
            pltpu.SemaphoreType.DMA,
        ],
        compiler_params=pltpu.CompilerParams(
            use_tc_tiling_on_sc=False, needs_layout_passes=False
        ),
    )
    def gather_kernel(idx_hbm, proj_hbm, out_hbm, idx2_v, idxf_v, rows_v, sem):
        wid = lax.axis_index("s") * _NC + lax.axis_index("c")
        base = wid * _BPW
        pltpu.sync_copy(idx_hbm.at[pl.ds(wid * _BROWS, _BROWS)], idx2_v)

        lanes = lax.iota(jnp.int32, _L)
        for g in range(_BPW // _L):
            jv = g * _L + lanes
            r = lax.shift_right_logical(jv * _MAGIC, _SHIFT)
            c = jv - r * _HIST
            idxf_v[pl.ds(g * _L, _L)] = plsc.load_gather(idx2_v, [r, c])

        for ch in range(_NCH):
            off = base + ch * _CH
            pltpu.async_copy(
                proj_hbm.at[idxf_v.at[pl.ds(ch * _CH, _CH)]], rows_v, sem
            ).wait()
            pltpu.sync_copy(rows_v, out_hbm.at[pl.ds(off, _CH)])

    return gather_kernel(idx_pad, proj)


def kernel(inputs, embeddings, project_kernel):
    idx_pad = jnp.pad(inputs.astype(jnp.int32), ((0, 0), (0, _DOUT - _HIST)))
    proj = _tc_project_table(embeddings, project_kernel)
    out = _sc_gather(idx_pad, proj)
    return out.reshape(_BATCH, _HIST, _DOUT)


# speedup vs baseline: 2.2553x; 1.5673x over previous
import functools

import jax
import jax.numpy as jnp
from jax import lax
from jax.experimental import pallas as pl
from jax.experimental.pallas import tpu as pltpu
from jax.experimental.pallas import tpu_sc as plsc

_BATCH = 4096
_HIST = 50
_BT = _BATCH * _HIST
_V = 1000000
_D = 32
_DOUT = 128

_NC = 2
_NS = 16
_NW = _NC * _NS
_BPW = _BT // _NW
_BROWS = _BATCH // _NW
_HPC = 5
_CH = _HPC * _BROWS
_NCH = _HIST // _HPC
_L = 16


def _tc_project_table(embeddings, project_kernel):
    tableT = jnp.transpose(embeddings)
    blk = 8192

    def body(tT_ref, p_ref, o_ref):
        o_ref[...] = jax.lax.dot_general(
            tT_ref[...], p_ref[...],
            dimension_numbers=(((0,), (0,)), ((), ())),
            preferred_element_type=jnp.float32,
        )

    return pl.pallas_call(
        body,
        grid=(pl.cdiv(_V, blk),),
        in_specs=[
            pl.BlockSpec((_D, blk), lambda i: (0, i)),
            pl.BlockSpec((_D, _DOUT), lambda i: (0, 0)),
        ],
        out_specs=pl.BlockSpec((blk, _DOUT), lambda i: (i, 0)),
        out_shape=jax.ShapeDtypeStruct((_V, _DOUT), jnp.float32),
    )(tableT, project_kernel)


def _sc_gather(idx_pad, proj):
    mesh = plsc.VectorSubcoreMesh(core_axis_name="c", subcore_axis_name="s")

    @functools.partial(
        pl.kernel,
        out_type=jax.ShapeDtypeStruct((_BT, _DOUT), jnp.float32),
        mesh=mesh,
        scratch_types=[
            pltpu.VMEM((_BROWS, _DOUT), jnp.int32),
            pltpu.VMEM((_BPW,), jnp.int32),
            pltpu.VMEM((_CH, _DOUT), jnp.float32),
            pltpu.SemaphoreType.DMA,
        ],
        compiler_params=pltpu.CompilerParams(
            use_tc_tiling_on_sc=False, needs_layout_passes=False
        ),
    )
    def gather_kernel(idx_hbm, proj_hbm, out_hbm, idx2_v, idxf_v, rows_v, sem):
        wid = lax.axis_index("s") * _NC + lax.axis_index("c")
        pltpu.sync_copy(idx_hbm.at[pl.ds(wid * _BROWS, _BROWS)], idx2_v)

        lanes = lax.iota(jnp.int32, _L)
        for g in range(_BPW // _L):
            q = g * _L + lanes
            h = lax.shift_right_logical(q, 7)
            b = lax.bitwise_and(q, 127)
            idxf_v[pl.ds(g * _L, _L)] = plsc.load_gather(idx2_v, [b, h])

        for ch in range(_NCH):
            pltpu.async_copy(
                proj_hbm.at[idxf_v.at[pl.ds(ch * _CH, _CH)]], rows_v, sem
            ).wait()
            for hh in range(_HPC):
                hplane = ch * _HPC + hh
                pltpu.sync_copy(
                    rows_v.at[pl.ds(hh * _BROWS, _BROWS)],
                    out_hbm.at[pl.ds(hplane * _BATCH + wid * _BROWS, _BROWS)],
                )

    return gather_kernel(idx_pad, proj)


def kernel(inputs, embeddings, project_kernel):
    idx_pad = jnp.pad(inputs.astype(jnp.int32), ((0, 0), (0, _DOUT - _HIST)))
    proj = _tc_project_table(embeddings, project_kernel)
    out = _sc_gather(idx_pad, proj)
    return out.reshape(_HIST, _BATCH, _DOUT).transpose(1, 0, 2)


# speedup vs baseline: 2.2872x; 1.0142x over previous
import functools

import jax
import jax.numpy as jnp
from jax import lax
from jax.experimental import pallas as pl
from jax.experimental.pallas import tpu as pltpu
from jax.experimental.pallas import tpu_sc as plsc

_BATCH = 4096
_HIST = 50
_BT = _BATCH * _HIST
_V = 1000000
_D = 32
_DOUT = 128

_NC = 2
_NS = 16
_NW = _NC * _NS
_BPW = _BT // _NW
_BROWS = _BATCH // _NW
_HPC = 2
_CH = _HPC * _BROWS
_NCH = _HIST // _HPC
_L = 16


def _tc_project_table(embeddings, project_kernel):
    tableT = jnp.transpose(embeddings)
    blk = 8192

    def body(tT_ref, p_ref, o_ref):
        o_ref[...] = jax.lax.dot_general(
            tT_ref[...], p_ref[...],
            dimension_numbers=(((0,), (0,)), ((), ())),
            preferred_element_type=jnp.float32,
        )

    return pl.pallas_call(
        body,
        grid=(pl.cdiv(_V, blk),),
        in_specs=[
            pl.BlockSpec((_D, blk), lambda i: (0, i)),
            pl.BlockSpec((_D, _DOUT), lambda i: (0, 0)),
        ],
        out_specs=pl.BlockSpec((blk, _DOUT), lambda i: (i, 0)),
        out_shape=jax.ShapeDtypeStruct((_V, _DOUT), jnp.float32),
    )(tableT, project_kernel)


def _sc_gather(idx_pad, proj):
    mesh = plsc.VectorSubcoreMesh(core_axis_name="c", subcore_axis_name="s")

    @functools.partial(
        pl.kernel,
        out_type=jax.ShapeDtypeStruct((_BT, _DOUT), jnp.float32),
        mesh=mesh,
        scratch_types=[
            pltpu.VMEM((_BROWS, _DOUT), jnp.int32),
            pltpu.VMEM((_BPW,), jnp.int32),
            pltpu.VMEM((_CH, _DOUT), jnp.float32),
            pltpu.VMEM((_CH, _DOUT), jnp.float32),
            pltpu.SemaphoreType.DMA,
            pltpu.SemaphoreType.DMA,
        ],
        compiler_params=pltpu.CompilerParams(
            use_tc_tiling_on_sc=False, needs_layout_passes=False
        ),
    )
    def gather_kernel(
        idx_hbm, proj_hbm, out_hbm, idx2_v, idxf_v, rows_v0, rows_v1, sem0, sem1
    ):
        wid = lax.axis_index("s") * _NC + lax.axis_index("c")
        pltpu.sync_copy(idx_hbm.at[pl.ds(wid * _BROWS, _BROWS)], idx2_v)

        lanes = lax.iota(jnp.int32, _L)
        for g in range(_BPW // _L):
            q = g * _L + lanes
            h = lax.shift_right_logical(q, 7)
            b = lax.bitwise_and(q, 127)
            idxf_v[pl.ds(g * _L, _L)] = plsc.load_gather(idx2_v, [b, h])

        bufs = (rows_v0, rows_v1)
        sems = (sem0, sem1)
        copies = [
            pltpu.async_copy(
                proj_hbm.at[idxf_v.at[pl.ds(ch * _CH, _CH)]],
                bufs[ch % 2],
                sems[ch % 2],
            )
            for ch in range(1)
        ]
        for ch in range(_NCH):
            if ch + 1 < _NCH:
                copies.append(
                    pltpu.async_copy(
                        proj_hbm.at[idxf_v.at[pl.ds((ch + 1) * _CH, _CH)]],
                        bufs[(ch + 1) % 2],
                        sems[(ch + 1) % 2],
                    )
                )
            copies[ch].wait()
            rows_v = bufs[ch % 2]
            for hh in range(_HPC):
                hplane = ch * _HPC + hh
                pltpu.sync_copy(
                    rows_v.at[pl.ds(hh * _BROWS, _BROWS)],
                    out_hbm.at[pl.ds(hplane * _BATCH + wid * _BROWS, _BROWS)],
                )

    return gather_kernel(idx_pad, proj)


def kernel(inputs, embeddings, project_kernel):
    idx_pad = jnp.pad(inputs.astype(jnp.int32), ((0, 0), (0, _DOUT - _HIST)))
    proj = _tc_project_table(embeddings, project_kernel)
    out = _sc_gather(idx_pad, proj)
    return out.reshape(_HIST, _BATCH, _DOUT).transpose(1, 0, 2)
